# baseline (speedup 1.0000x reference)
import functools

import jax
import jax.numpy as jnp
from jax import lax
from jax.experimental import pallas as pl
from jax.experimental.pallas import tpu as pltpu
from jax.experimental.pallas import tpu_sc as plsc

NC = 2
NS = 16
NW = NC * NS
L = 16
CH = 128
NB = 3
GD = 2


def _sc_mesh():
    return plsc.VectorSubcoreMesh(core_axis_name="c", subcore_axis_name="s",
                                  num_cores=NC, num_subcores=NS)


_SC_PARAMS = pltpu.CompilerParams(needs_layout_passes=False,
                                  use_tc_tiling_on_sc=False)


def _make_deg_kernel(N, NPAD, EPW):
    @functools.partial(
        pl.kernel,
        out_type=jax.ShapeDtypeStruct((NW, NPAD), jnp.float32),
        mesh=_sc_mesh(),
        compiler_params=_SC_PARAMS,
        scratch_types=[
            pltpu.VMEM((EPW,), jnp.int32),
            pltpu.VMEM((NPAD,), jnp.float32),
        ],
    )
    def deg_kernel(dst_hbm, out_hbm, idx_v, acc_v):
        cid = lax.axis_index("c")
        sid = lax.axis_index("s")
        w = cid * NS + sid
        pltpu.sync_copy(dst_hbm.at[w], idx_v)

        @pl.loop(0, NPAD // L)
        def _zero(i):
            acc_v[pl.ds(i * L, L)] = jnp.zeros((L,), jnp.float32)

        ones = jnp.ones((L,), jnp.float32)

        @pl.loop(0, EPW // L)
        def _accum(i):
            idx = idx_v[pl.ds(i * L, L)]
            plsc.addupdate_scatter(acc_v, [idx], ones)

        pltpu.sync_copy(acc_v, out_hbm.at[w])

    return deg_kernel


def _make_hop_kernel(N, NPAD, D, CPW):
    rpt = NPAD // NS

    @functools.partial(
        pl.kernel,
        out_type=jax.ShapeDtypeStruct((NC, NPAD, D), jnp.float32),
        mesh=_sc_mesh(),
        compiler_params=_SC_PARAMS,
        scratch_types=[
            pltpu.VMEM((CPW, CH), jnp.int32),
            pltpu.VMEM((CPW, CH), jnp.int32),
            [pltpu.VMEM((CH, D), jnp.float32) for _ in range(NB)],
            pltpu.VMEM_SHARED((NPAD, D), jnp.float32),
            pltpu.VMEM_SHARED((NPAD, D), jnp.float32),
            pltpu.SemaphoreType.DMA((NB,)),
            pltpu.SemaphoreType.DMA((NB,)),
        ],
    )
    def hop_kernel(y_hbm, src_hbm, dst_hbm, zeros_hbm, out_hbm,
                   src_v, dst_v, rbufs, y_sh, acc_sh, gsem, ssem):
        cid = lax.axis_index("c")
        sid = lax.axis_index("s")
        w = cid * NS + sid

        row0 = pl.multiple_of(sid * rpt, 8)
        pltpu.sync_copy(zeros_hbm.at[pl.ds(row0, rpt)],
                        acc_sh.at[pl.ds(row0, rpt)])
        pltpu.sync_copy(y_hbm.at[pl.ds(row0, rpt)],
                        y_sh.at[pl.ds(row0, rpt)])
        pltpu.sync_copy(src_hbm.at[w], src_v)
        pltpu.sync_copy(dst_hbm.at[w], dst_v)
        plsc.subcore_barrier()

        for b in range(GD):
            pltpu.async_copy(y_sh.at[src_v.at[b]], rbufs[b], gsem.at[b])

        @pl.loop(0, CPW // NB)
        def _chunks(i):
            for b in range(NB):
                k = i * NB + b
                bs = (b - (NB - GD)) % NB

                @pl.when(k >= NB - GD)
                def _():
                    ks = k - (NB - GD)
                    pltpu.make_async_copy(rbufs[bs], acc_sh.at[dst_v.at[ks]],
                                          ssem.at[bs]).wait()

                @pl.when(k + GD < CPW)
                def _():
                    bg = (b + GD) % NB
                    pltpu.async_copy(y_sh.at[src_v.at[k + GD]], rbufs[bg],
                                     gsem.at[bg])

                pltpu.make_async_copy(y_sh.at[src_v.at[k]], rbufs[b],
                                      gsem.at[b]).wait()
                pltpu.async_copy(rbufs[b], acc_sh.at[dst_v.at[k]], ssem.at[b],
                                 add=True)

        for j in range(CPW - (NB - GD), CPW):
            b = j % NB
            pltpu.make_async_copy(rbufs[b], acc_sh.at[dst_v.at[j]],
                                  ssem.at[b]).wait()

        plsc.subcore_barrier()
        pltpu.sync_copy(acc_sh.at[pl.ds(row0, rpt)],
                        out_hbm.at[cid, pl.ds(row0, rpt)])

    return hop_kernel


def _tc_a(x, W, degt, NPAD, BR):
    N, DIN = x.shape
    DOUT = W.shape[1]

    def body(x_ref, w_ref, d_ref, z_ref, n_ref):
        deg = jnp.sum(d_ref[...], axis=1, keepdims=True)
        deg = jnp.maximum(deg, 1.0)
        nrm = lax.rsqrt(deg)
        y = jnp.dot(x_ref[...], w_ref[...], preferred_element_type=jnp.float32)
        z_ref[...] = y * nrm
        n_ref[...] = nrm

    return pl.pallas_call(
        body,
        grid=(NPAD // BR,),
        in_specs=[
            pl.BlockSpec((BR, DIN), lambda i: (i, 0)),
            pl.BlockSpec((DIN, DOUT), lambda i: (0, 0)),
            pl.BlockSpec((BR, NW), lambda i: (i, 0)),
        ],
        out_specs=[
            pl.BlockSpec((BR, DOUT), lambda i: (i, 0)),
            pl.BlockSpec((BR, 1), lambda i: (i, 0)),
        ],
        out_shape=[
            jax.ShapeDtypeStruct((NPAD, DOUT), jnp.float32),
            jax.ShapeDtypeStruct((NPAD, 1), jnp.float32),
        ],
    )(x, W, degt)


def _tc_combine(p, nrm, BR, power, relu, out_rows):
    _, NPAD, D = p.shape

    def body(p_ref, n_ref, o_ref):
        s = p_ref[0] + p_ref[1]
        n = n_ref[...]
        scale = n * n if power == 2 else n
        z = s * scale
        if relu:
            z = jnp.maximum(z, 0.0)
        o_ref[...] = z

    return pl.pallas_call(
        body,
        grid=(NPAD // BR,),
        in_specs=[
            pl.BlockSpec((NC, BR, D), lambda i: (0, i, 0)),
            pl.BlockSpec((BR, 1), lambda i: (i, 0)),
        ],
        out_specs=pl.BlockSpec((BR, D), lambda i: (i, 0)),
        out_shape=jax.ShapeDtypeStruct((out_rows, D), jnp.float32),
    )(p, nrm)


def kernel(x, edge_index, W):
    N, DIN = x.shape
    DOUT = W.shape[1]
    E = edge_index.shape[1]
    NPAD = -(-(N + 1) // (8 * NS)) * (8 * NS)

    epw0 = -(-E // NW)
    CPW = -(-epw0 // CH)
    CPW = -(-CPW // NB) * NB
    EPW = CPW * CH
    EPAD = NW * EPW

    src = edge_index[0]
    dst = edge_index[1]
    pad = EPAD - E
    srcp = jnp.concatenate([src, jnp.zeros((pad,), jnp.int32)]).reshape(NW, CPW, CH)
    dstp = jnp.concatenate([dst, jnp.full((pad,), N, jnp.int32)]).reshape(NW, CPW, CH)
    dst_flat = dstp.reshape(NW, EPW)
    zeros = jnp.zeros((NPAD, DOUT), jnp.float32)

    degp = _make_deg_kernel(N, NPAD, EPW)(dst_flat)
    degt = degp.T

    BR = NPAD // NS
    z0, nrm = _tc_a(x, W, degt, NPAD, BR)

    hop = _make_hop_kernel(N, NPAD, DOUT, CPW)
    p1 = hop(z0, srcp, dstp, zeros)
    z1 = _tc_combine(p1, nrm, BR, power=2, relu=False, out_rows=NPAD)
    p2 = hop(z1, srcp, dstp, zeros)
    return _tc_combine(p2, nrm, BR, power=1, relu=True, out_rows=N)


# speedup vs baseline: 1.0694x; 1.0694x over previous
import functools

import jax
import jax.numpy as jnp
from jax import lax
from jax.experimental import pallas as pl
from jax.experimental.pallas import tpu as pltpu
from jax.experimental.pallas import tpu_sc as plsc

NC = 2
NS = 16
NW = NC * NS
L = 16
CH = 128
NB = 3
GD = 1


def _sc_mesh():
    return plsc.VectorSubcoreMesh(core_axis_name="c", subcore_axis_name="s",
                                  num_cores=NC, num_subcores=NS)


_SC_PARAMS = pltpu.CompilerParams(needs_layout_passes=False,
                                  use_tc_tiling_on_sc=False)


def _make_deg_kernel(N, NPAD, EPW):
    @functools.partial(
        pl.kernel,
        out_type=jax.ShapeDtypeStruct((NW, NPAD), jnp.float32),
        mesh=_sc_mesh(),
        compiler_params=_SC_PARAMS,
        scratch_types=[
            pltpu.VMEM((EPW,), jnp.int32),
            pltpu.VMEM((NPAD,), jnp.float32),
        ],
    )
    def deg_kernel(dst_hbm, out_hbm, idx_v, acc_v):
        cid = lax.axis_index("c")
        sid = lax.axis_index("s")
        w = cid * NS + sid
        pltpu.sync_copy(dst_hbm.at[w], idx_v)

        @pl.loop(0, NPAD // L)
        def _zero(i):
            acc_v[pl.ds(i * L, L)] = jnp.zeros((L,), jnp.float32)

        ones = jnp.ones((L,), jnp.float32)

        @pl.loop(0, EPW // L)
        def _accum(i):
            idx = idx_v[pl.ds(i * L, L)]
            plsc.addupdate_scatter(acc_v, [idx], ones)

        pltpu.sync_copy(acc_v, out_hbm.at[w])

    return deg_kernel


def _make_hop_kernel(N, NPAD, D, CPW):
    rpt = NPAD // NS

    @functools.partial(
        pl.kernel,
        out_type=jax.ShapeDtypeStruct((NC, NPAD, D), jnp.float32),
        mesh=_sc_mesh(),
        compiler_params=_SC_PARAMS,
        scratch_types=[
            pltpu.VMEM((CPW, CH), jnp.int32),
            pltpu.VMEM((CPW, CH), jnp.int32),
            [pltpu.VMEM((CH, D), jnp.float32) for _ in range(NB)],
            pltpu.VMEM_SHARED((NPAD, D), jnp.float32),
            pltpu.VMEM_SHARED((NPAD, D), jnp.float32),
            pltpu.SemaphoreType.DMA((NB,)),
            pltpu.SemaphoreType.DMA((NB,)),
        ],
    )
    def hop_kernel(y_hbm, src_hbm, dst_hbm, zeros_hbm, out_hbm,
                   src_v, dst_v, rbufs, y_sh, acc_sh, gsem, ssem):
        cid = lax.axis_index("c")
        sid = lax.axis_index("s")
        w = cid * NS + sid

        row0 = pl.multiple_of(sid * rpt, 8)
        pltpu.sync_copy(zeros_hbm.at[pl.ds(row0, rpt)],
                        acc_sh.at[pl.ds(row0, rpt)])
        pltpu.sync_copy(y_hbm.at[pl.ds(row0, rpt)],
                        y_sh.at[pl.ds(row0, rpt)])
        pltpu.sync_copy(src_hbm.at[w], src_v)
        pltpu.sync_copy(dst_hbm.at[w], dst_v)
        plsc.subcore_barrier()

        for b in range(GD):
            pltpu.async_copy(y_sh.at[src_v.at[b]], rbufs[b], gsem.at[b])

        @pl.loop(0, CPW // NB)
        def _chunks(i):
            for b in range(NB):
                k = i * NB + b
                bs = (b - (NB - GD)) % NB

                @pl.when(k >= NB - GD)
                def _():
                    ks = k - (NB - GD)
                    pltpu.make_async_copy(rbufs[bs], acc_sh.at[dst_v.at[ks]],
                                          ssem.at[bs]).wait()

                @pl.when(k + GD < CPW)
                def _():
                    bg = (b + GD) % NB
                    pltpu.async_copy(y_sh.at[src_v.at[k + GD]], rbufs[bg],
                                     gsem.at[bg])

                pltpu.make_async_copy(y_sh.at[src_v.at[k]], rbufs[b],
                                      gsem.at[b]).wait()
                pltpu.async_copy(rbufs[b], acc_sh.at[dst_v.at[k]], ssem.at[b],
                                 add=True)

        for j in range(CPW - (NB - GD), CPW):
            b = j % NB
            pltpu.make_async_copy(rbufs[b], acc_sh.at[dst_v.at[j]],
                                  ssem.at[b]).wait()

        plsc.subcore_barrier()
        pltpu.sync_copy(acc_sh.at[pl.ds(row0, rpt)],
                        out_hbm.at[cid, pl.ds(row0, rpt)])

    return hop_kernel


def _tc_a(x, W, degt, NPAD, BR):
    N, DIN = x.shape
    DOUT = W.shape[1]

    def body(x_ref, w_ref, d_ref, z_ref, n_ref):
        deg = jnp.sum(d_ref[...], axis=1, keepdims=True)
        deg = jnp.maximum(deg, 1.0)
        nrm = lax.rsqrt(deg)
        y = jnp.dot(x_ref[...], w_ref[...], preferred_element_type=jnp.float32)
        z_ref[...] = y * nrm
        n_ref[...] = nrm

    return pl.pallas_call(
        body,
        grid=(NPAD // BR,),
        in_specs=[
            pl.BlockSpec((BR, DIN), lambda i: (i, 0)),
            pl.BlockSpec((DIN, DOUT), lambda i: (0, 0)),
            pl.BlockSpec((BR, NW), lambda i: (i, 0)),
        ],
        out_specs=[
            pl.BlockSpec((BR, DOUT), lambda i: (i, 0)),
            pl.BlockSpec((BR, 1), lambda i: (i, 0)),
        ],
        out_shape=[
            jax.ShapeDtypeStruct((NPAD, DOUT), jnp.float32),
            jax.ShapeDtypeStruct((NPAD, 1), jnp.float32),
        ],
    )(x, W, degt)


def _tc_combine(p, nrm, BR, power, relu, out_rows):
    _, NPAD, D = p.shape

    def body(p_ref, n_ref, o_ref):
        s = p_ref[0] + p_ref[1]
        n = n_ref[...]
        scale = n * n if power == 2 else n
        z = s * scale
        if relu:
            z = jnp.maximum(z, 0.0)
        o_ref[...] = z

    return pl.pallas_call(
        body,
        grid=(NPAD // BR,),
        in_specs=[
            pl.BlockSpec((NC, BR, D), lambda i: (0, i, 0)),
            pl.BlockSpec((BR, 1), lambda i: (i, 0)),
        ],
        out_specs=pl.BlockSpec((BR, D), lambda i: (i, 0)),
        out_shape=jax.ShapeDtypeStruct((out_rows, D), jnp.float32),
    )(p, nrm)


def kernel(x, edge_index, W):
    N, DIN = x.shape
    DOUT = W.shape[1]
    E = edge_index.shape[1]
    NPAD = -(-(N + 1) // (8 * NS)) * (8 * NS)

    epw0 = -(-E // NW)
    CPW = -(-epw0 // CH)
    CPW = -(-CPW // NB) * NB
    EPW = CPW * CH
    EPAD = NW * EPW

    src = edge_index[0]
    dst = edge_index[1]
    pad = EPAD - E
    srcp = jnp.concatenate([src, jnp.zeros((pad,), jnp.int32)]).reshape(NW, CPW, CH)
    dstp = jnp.concatenate([dst, jnp.full((pad,), N, jnp.int32)]).reshape(NW, CPW, CH)
    dst_flat = dstp.reshape(NW, EPW)
    zeros = jnp.zeros((NPAD, DOUT), jnp.float32)

    degp = _make_deg_kernel(N, NPAD, EPW)(dst_flat)
    degt = degp.T

    BR = NPAD // NS
    z0, nrm = _tc_a(x, W, degt, NPAD, BR)

    hop = _make_hop_kernel(N, NPAD, DOUT, CPW)
    p1 = hop(z0, srcp, dstp, zeros)
    z1 = _tc_combine(p1, nrm, BR, power=2, relu=False, out_rows=NPAD)
    p2 = hop(z1, srcp, dstp, zeros)
    return _tc_combine(p2, nrm, BR, power=1, relu=True, out_rows=N)


# speedup vs baseline: 1.0732x; 1.0035x over previous
import functools

import jax
import jax.numpy as jnp
from jax import lax
from jax.experimental import pallas as pl
from jax.experimental.pallas import tpu as pltpu
from jax.experimental.pallas import tpu_sc as plsc

NC = 2
NS = 16
NW = NC * NS
L = 16
CH = 112
NB = 4
GD = 1


def _sc_mesh():
    return plsc.VectorSubcoreMesh(core_axis_name="c", subcore_axis_name="s",
                                  num_cores=NC, num_subcores=NS)


_SC_PARAMS = pltpu.CompilerParams(needs_layout_passes=False,
                                  use_tc_tiling_on_sc=False)


def _make_deg_kernel(N, NPAD, EPW):
    @functools.partial(
        pl.kernel,
        out_type=jax.ShapeDtypeStruct((NW, NPAD), jnp.float32),
        mesh=_sc_mesh(),
        compiler_params=_SC_PARAMS,
        scratch_types=[
            pltpu.VMEM((EPW,), jnp.int32),
            pltpu.VMEM((NPAD,), jnp.float32),
        ],
    )
    def deg_kernel(dst_hbm, out_hbm, idx_v, acc_v):
        cid = lax.axis_index("c")
        sid = lax.axis_index("s")
        w = cid * NS + sid
        pltpu.sync_copy(dst_hbm.at[w], idx_v)

        @pl.loop(0, NPAD // L)
        def _zero(i):
            acc_v[pl.ds(i * L, L)] = jnp.zeros((L,), jnp.float32)

        ones = jnp.ones((L,), jnp.float32)

        @pl.loop(0, EPW // L)
        def _accum(i):
            idx = idx_v[pl.ds(i * L, L)]
            plsc.addupdate_scatter(acc_v, [idx], ones)

        pltpu.sync_copy(acc_v, out_hbm.at[w])

    return deg_kernel


def _make_hop_kernel(N, NPAD, D, CPW):
    rpt = NPAD // NS

    @functools.partial(
        pl.kernel,
        out_type=jax.ShapeDtypeStruct((NC, NPAD, D), jnp.float32),
        mesh=_sc_mesh(),
        compiler_params=_SC_PARAMS,
        scratch_types=[
            pltpu.VMEM((CPW, CH), jnp.int32),
            pltpu.VMEM((CPW, CH), jnp.int32),
            [pltpu.VMEM((CH, D), jnp.float32) for _ in range(NB)],
            pltpu.VMEM_SHARED((NPAD, D), jnp.float32),
            pltpu.VMEM_SHARED((NPAD, D), jnp.float32),
            pltpu.SemaphoreType.DMA((NB,)),
            pltpu.SemaphoreType.DMA((NB,)),
        ],
    )
    def hop_kernel(y_hbm, src_hbm, dst_hbm, zeros_hbm, out_hbm,
                   src_v, dst_v, rbufs, y_sh, acc_sh, gsem, ssem):
        cid = lax.axis_index("c")
        sid = lax.axis_index("s")
        w = cid * NS + sid

        row0 = pl.multiple_of(sid * rpt, 8)
        pltpu.sync_copy(zeros_hbm.at[pl.ds(row0, rpt)],
                        acc_sh.at[pl.ds(row0, rpt)])
        pltpu.sync_copy(y_hbm.at[pl.ds(row0, rpt)],
                        y_sh.at[pl.ds(row0, rpt)])
        pltpu.sync_copy(src_hbm.at[w], src_v)
        pltpu.sync_copy(dst_hbm.at[w], dst_v)
        plsc.subcore_barrier()

        for b in range(GD):
            pltpu.async_copy(y_sh.at[src_v.at[b]], rbufs[b], gsem.at[b])

        @pl.loop(0, CPW // NB)
        def _chunks(i):
            for b in range(NB):
                k = i * NB + b
                bs = (b - (NB - GD)) % NB

                @pl.when(k >= NB - GD)
                def _():
                    ks = k - (NB - GD)
                    pltpu.make_async_copy(rbufs[bs], acc_sh.at[dst_v.at[ks]],
                                          ssem.at[bs]).wait()

                @pl.when(k + GD < CPW)
                def _():
                    bg = (b + GD) % NB
                    pltpu.async_copy(y_sh.at[src_v.at[k + GD]], rbufs[bg],
                                     gsem.at[bg])

                pltpu.make_async_copy(y_sh.at[src_v.at[k]], rbufs[b],
                                      gsem.at[b]).wait()
                pltpu.async_copy(rbufs[b], acc_sh.at[dst_v.at[k]], ssem.at[b],
                                 add=True)

        for j in range(CPW - (NB - GD), CPW):
            b = j % NB
            pltpu.make_async_copy(rbufs[b], acc_sh.at[dst_v.at[j]],
                                  ssem.at[b]).wait()

        plsc.subcore_barrier()
        pltpu.sync_copy(acc_sh.at[pl.ds(row0, rpt)],
                        out_hbm.at[cid, pl.ds(row0, rpt)])

    return hop_kernel


def _tc_a(x, W, degt, NPAD, BR):
    N, DIN = x.shape
    DOUT = W.shape[1]

    def body(x_ref, w_ref, d_ref, z_ref, n_ref):
        deg = jnp.sum(d_ref[...], axis=1, keepdims=True)
        deg = jnp.maximum(deg, 1.0)
        nrm = lax.rsqrt(deg)
        y = jnp.dot(x_ref[...], w_ref[...], preferred_element_type=jnp.float32)
        z_ref[...] = y * nrm
        n_ref[...] = nrm

    return pl.pallas_call(
        body,
        grid=(NPAD // BR,),
        in_specs=[
            pl.BlockSpec((BR, DIN), lambda i: (i, 0)),
            pl.BlockSpec((DIN, DOUT), lambda i: (0, 0)),
            pl.BlockSpec((BR, NW), lambda i: (i, 0)),
        ],
        out_specs=[
            pl.BlockSpec((BR, DOUT), lambda i: (i, 0)),
            pl.BlockSpec((BR, 1), lambda i: (i, 0)),
        ],
        out_shape=[
            jax.ShapeDtypeStruct((NPAD, DOUT), jnp.float32),
            jax.ShapeDtypeStruct((NPAD, 1), jnp.float32),
        ],
    )(x, W, degt)


def _tc_combine(p, nrm, BR, power, relu, out_rows):
    _, NPAD, D = p.shape

    def body(p_ref, n_ref, o_ref):
        s = p_ref[0] + p_ref[1]
        n = n_ref[...]
        scale = n * n if power == 2 else n
        z = s * scale
        if relu:
            z = jnp.maximum(z, 0.0)
        o_ref[...] = z

    return pl.pallas_call(
        body,
        grid=(NPAD // BR,),
        in_specs=[
            pl.BlockSpec((NC, BR, D), lambda i: (0, i, 0)),
            pl.BlockSpec((BR, 1), lambda i: (i, 0)),
        ],
        out_specs=pl.BlockSpec((BR, D), lambda i: (i, 0)),
        out_shape=jax.ShapeDtypeStruct((out_rows, D), jnp.float32),
    )(p, nrm)


def kernel(x, edge_index, W):
    N, DIN = x.shape
    DOUT = W.shape[1]
    E = edge_index.shape[1]
    NPAD = -(-(N + 1) // (8 * NS)) * (8 * NS)

    epw0 = -(-E // NW)
    CPW = -(-epw0 // CH)
    CPW = -(-CPW // NB) * NB
    EPW = CPW * CH
    EPAD = NW * EPW

    src = edge_index[0]
    dst = edge_index[1]
    pad = EPAD - E
    srcp = jnp.concatenate([src, jnp.zeros((pad,), jnp.int32)]).reshape(NW, CPW, CH)
    dstp = jnp.concatenate([dst, jnp.full((pad,), N, jnp.int32)]).reshape(NW, CPW, CH)
    dst_flat = dstp.reshape(NW, EPW)
    zeros = jnp.zeros((NPAD, DOUT), jnp.float32)

    degp = _make_deg_kernel(N, NPAD, EPW)(dst_flat)
    degt = degp.T

    BR = NPAD // NS
    z0, nrm = _tc_a(x, W, degt, NPAD, BR)

    hop = _make_hop_kernel(N, NPAD, DOUT, CPW)
    p1 = hop(z0, srcp, dstp, zeros)
    z1 = _tc_combine(p1, nrm, BR, power=2, relu=False, out_rows=NPAD)
    p2 = hop(z1, srcp, dstp, zeros)
    return _tc_combine(p2, nrm, BR, power=1, relu=True, out_rows=N)


# speedup vs baseline: 1.1556x; 1.0768x over previous
import functools

import jax
import jax.numpy as jnp
from jax import lax
from jax.experimental import pallas as pl
from jax.experimental.pallas import tpu as pltpu
from jax.experimental.pallas import tpu_sc as plsc

NC = 2
NS = 16
NW = NC * NS
L = 16
CH = 112
NB = 4
GD = 1


def _sc_mesh():
    return plsc.VectorSubcoreMesh(core_axis_name="c", subcore_axis_name="s",
                                  num_cores=NC, num_subcores=NS)


_SC_PARAMS = pltpu.CompilerParams(needs_layout_passes=False,
                                  use_tc_tiling_on_sc=False)


def _make_deg_kernel(N, NPAD, EPW):
    @functools.partial(
        pl.kernel,
        out_type=jax.ShapeDtypeStruct((NW, NPAD), jnp.float32),
        mesh=_sc_mesh(),
        compiler_params=_SC_PARAMS,
        scratch_types=[
            pltpu.VMEM((EPW,), jnp.int32),
            pltpu.VMEM((NPAD,), jnp.float32),
        ],
    )
    def deg_kernel(dst_hbm, out_hbm, idx_v, acc_v):
        cid = lax.axis_index("c")
        sid = lax.axis_index("s")
        w = cid * NS + sid
        pltpu.sync_copy(dst_hbm.at[w], idx_v)

        @pl.loop(0, NPAD // L)
        def _zero(i):
            acc_v[pl.ds(i * L, L)] = jnp.zeros((L,), jnp.float32)

        ones = jnp.ones((L,), jnp.float32)

        @pl.loop(0, EPW // L)
        def _accum(i):
            idx = idx_v[pl.ds(i * L, L)]
            plsc.addupdate_scatter(acc_v, [idx], ones)

        pltpu.sync_copy(acc_v, out_hbm.at[w])

    return deg_kernel


def _make_hop_kernel(N, NPAD, D, CPW):
    rpt = NPAD // NS

    @functools.partial(
        pl.kernel,
        out_type=jax.ShapeDtypeStruct((NC, NPAD, D), jnp.float32),
        mesh=_sc_mesh(),
        compiler_params=_SC_PARAMS,
        scratch_types=[
            pltpu.VMEM((CPW, CH), jnp.int32),
            pltpu.VMEM((CPW, CH), jnp.int32),
            [pltpu.VMEM((CH, D), jnp.float32) for _ in range(NB)],
            pltpu.VMEM_SHARED((NPAD, D), jnp.float32),
            pltpu.VMEM_SHARED((NPAD, D), jnp.float32),
            pltpu.SemaphoreType.DMA((NB,)),
            pltpu.SemaphoreType.DMA((NB,)),
        ],
    )
    def hop_kernel(y_hbm, src_hbm, dst_hbm, zeros_hbm, out_hbm,
                   src_v, dst_v, rbufs, y_sh, acc_sh, gsem, ssem):
        cid = lax.axis_index("c")
        sid = lax.axis_index("s")
        w = cid * NS + sid

        row0 = pl.multiple_of(sid * rpt, 8)
        pltpu.sync_copy(zeros_hbm.at[pl.ds(row0, rpt)],
                        acc_sh.at[pl.ds(row0, rpt)])
        pltpu.sync_copy(y_hbm.at[pl.ds(row0, rpt)],
                        y_sh.at[pl.ds(row0, rpt)])
        pltpu.sync_copy(src_hbm.at[w], src_v)
        pltpu.sync_copy(dst_hbm.at[w], dst_v)
        plsc.subcore_barrier()

        for b in range(GD):
            pltpu.async_copy(y_sh.at[src_v.at[b]], rbufs[b], gsem.at[b])

        @pl.loop(0, CPW // NB)
        def _chunks(i):
            for b in range(NB):
                k = i * NB + b
                bs = (b - (NB - GD)) % NB

                @pl.when(k >= NB - GD)
                def _():
                    ks = k - (NB - GD)
                    pltpu.make_async_copy(rbufs[bs], acc_sh.at[dst_v.at[ks]],
                                          ssem.at[bs]).wait()

                @pl.when(k + GD < CPW)
                def _():
                    bg = (b + GD) % NB
                    pltpu.async_copy(y_sh.at[src_v.at[k + GD]], rbufs[bg],
                                     gsem.at[bg])

                pltpu.make_async_copy(y_sh.at[src_v.at[k]], rbufs[b],
                                      gsem.at[b]).wait()
                pltpu.async_copy(rbufs[b], acc_sh.at[dst_v.at[k]], ssem.at[b],
                                 add=True)

        for j in range(CPW - (NB - GD), CPW):
            b = j % NB
            pltpu.make_async_copy(rbufs[b], acc_sh.at[dst_v.at[j]],
                                  ssem.at[b]).wait()

        plsc.subcore_barrier()
        pltpu.sync_copy(acc_sh.at[pl.ds(row0, rpt)],
                        out_hbm.at[cid, pl.ds(row0, rpt)])

    return hop_kernel


def _tc_a(x, W, degt, NPAD, BR):
    N, DIN = x.shape
    DOUT = W.shape[1]

    def body(x_ref, w_ref, d_ref, z_ref, n_ref):
        deg = jnp.sum(d_ref[...], axis=0, keepdims=True)
        deg = jnp.maximum(deg, 1.0)
        nrm = jnp.transpose(lax.rsqrt(deg))
        y = jnp.dot(x_ref[...], w_ref[...], preferred_element_type=jnp.float32)
        npad = n_ref.shape[0]
        ypad = jnp.pad(y, ((0, npad - y.shape[0]), (0, 0)))
        z_ref[...] = ypad * nrm
        n_ref[...] = nrm

    return pl.pallas_call(
        body,
        out_shape=[
            jax.ShapeDtypeStruct((NPAD, DOUT), jnp.float32),
            jax.ShapeDtypeStruct((NPAD, 1), jnp.float32),
        ],
    )(x, W, degt)


def _tc_combine(p, nrm, BR, power, relu, out_rows):
    _, NPAD, D = p.shape

    def body(p_ref, n_ref, o_ref):
        s = p_ref[0] + p_ref[1]
        n = n_ref[...]
        scale = n * n if power == 2 else n
        z = s * scale
        if relu:
            z = jnp.maximum(z, 0.0)
        o_ref[...] = z

    def body2(p_ref, n_ref, o_ref):
        s = p_ref[0, :out_rows] + p_ref[1, :out_rows]
        n = n_ref[:out_rows]
        scale = n * n if power == 2 else n
        z = s * scale
        if relu:
            z = jnp.maximum(z, 0.0)
        o_ref[...] = z

    return pl.pallas_call(
        body2,
        out_shape=jax.ShapeDtypeStruct((out_rows, D), jnp.float32),
    )(p, nrm)


def kernel(x, edge_index, W):
    N, DIN = x.shape
    DOUT = W.shape[1]
    E = edge_index.shape[1]
    NPAD = -(-(N + 1) // (8 * NS)) * (8 * NS)

    epw0 = -(-E // NW)
    CPW = -(-epw0 // CH)
    CPW = -(-CPW // NB) * NB
    EPW = CPW * CH
    EPAD = NW * EPW

    src = edge_index[0]
    dst = edge_index[1]
    pad = EPAD - E
    srcp = jnp.concatenate([src, jnp.zeros((pad,), jnp.int32)]).reshape(NW, CPW, CH)
    dstp = jnp.concatenate([dst, jnp.full((pad,), N, jnp.int32)]).reshape(NW, CPW, CH)
    dst_flat = dstp.reshape(NW, EPW)
    zeros = jnp.zeros((NPAD, DOUT), jnp.float32)

    degp = _make_deg_kernel(N, NPAD, EPW)(dst_flat)

    BR = NPAD // NS
    z0, nrm = _tc_a(x, W, degp, NPAD, BR)

    hop = _make_hop_kernel(N, NPAD, DOUT, CPW)
    p1 = hop(z0, srcp, dstp, zeros)
    z1 = _tc_combine(p1, nrm, BR, power=2, relu=False, out_rows=NPAD)
    p2 = hop(z1, srcp, dstp, zeros)
    return _tc_combine(p2, nrm, BR, power=1, relu=True, out_rows=N)


# speedup vs baseline: 1.1627x; 1.0061x over previous
import functools

import jax
import jax.numpy as jnp
from jax import lax
from jax.experimental import pallas as pl
from jax.experimental.pallas import tpu as pltpu
from jax.experimental.pallas import tpu_sc as plsc

NC = 2
NS = 16
NW = NC * NS
L = 16
CH = 112
NB = 4
GD = 2


def _sc_mesh():
    return plsc.VectorSubcoreMesh(core_axis_name="c", subcore_axis_name="s",
                                  num_cores=NC, num_subcores=NS)


_SC_PARAMS = pltpu.CompilerParams(needs_layout_passes=False,
                                  use_tc_tiling_on_sc=False)


def _make_deg_kernel(N, NPAD, EPW):
    @functools.partial(
        pl.kernel,
        out_type=jax.ShapeDtypeStruct((NW, NPAD), jnp.float32),
        mesh=_sc_mesh(),
        compiler_params=_SC_PARAMS,
        scratch_types=[
            pltpu.VMEM((EPW,), jnp.int32),
            pltpu.VMEM((NPAD,), jnp.float32),
        ],
    )
    def deg_kernel(dst_hbm, out_hbm, idx_v, acc_v):
        cid = lax.axis_index("c")
        sid = lax.axis_index("s")
        w = cid * NS + sid
        pltpu.sync_copy(dst_hbm.at[w], idx_v)

        @pl.loop(0, NPAD // L, unroll=4)
        def _zero(i):
            acc_v[pl.ds(i * L, L)] = jnp.zeros((L,), jnp.float32)

        ones = jnp.ones((L,), jnp.float32)

        @pl.loop(0, EPW // L, unroll=4)
        def _accum(i):
            idx = idx_v[pl.ds(i * L, L)]
            plsc.addupdate_scatter(acc_v, [idx], ones)

        pltpu.sync_copy(acc_v, out_hbm.at[w])

    return deg_kernel


def _make_hop_kernel(N, NPAD, D, CPW):
    rpt = NPAD // NS

    @functools.partial(
        pl.kernel,
        out_type=jax.ShapeDtypeStruct((NC, NPAD, D), jnp.float32),
        mesh=_sc_mesh(),
        compiler_params=_SC_PARAMS,
        scratch_types=[
            pltpu.VMEM((CPW, CH), jnp.int32),
            pltpu.VMEM((CPW, CH), jnp.int32),
            [pltpu.VMEM((CH, D), jnp.float32) for _ in range(NB)],
            pltpu.VMEM_SHARED((NPAD, D), jnp.float32),
            pltpu.VMEM_SHARED((NPAD, D), jnp.float32),
            pltpu.SemaphoreType.DMA((NB,)),
            pltpu.SemaphoreType.DMA((NB,)),
        ],
    )
    def hop_kernel(y_hbm, src_hbm, dst_hbm, zeros_hbm, out_hbm,
                   src_v, dst_v, rbufs, y_sh, acc_sh, gsem, ssem):
        cid = lax.axis_index("c")
        sid = lax.axis_index("s")
        w = cid * NS + sid

        row0 = pl.multiple_of(sid * rpt, 8)
        pltpu.sync_copy(zeros_hbm.at[pl.ds(row0, rpt)],
                        acc_sh.at[pl.ds(row0, rpt)])
        pltpu.sync_copy(y_hbm.at[pl.ds(row0, rpt)],
                        y_sh.at[pl.ds(row0, rpt)])
        pltpu.sync_copy(src_hbm.at[w], src_v)
        pltpu.sync_copy(dst_hbm.at[w], dst_v)
        plsc.subcore_barrier()

        for b in range(GD):
            pltpu.async_copy(y_sh.at[src_v.at[b]], rbufs[b], gsem.at[b])

        @pl.loop(0, CPW // NB)
        def _chunks(i):
            for b in range(NB):
                k = i * NB + b
                bs = (b - (NB - GD)) % NB

                @pl.when(k >= NB - GD)
                def _():
                    ks = k - (NB - GD)
                    pltpu.make_async_copy(rbufs[bs], acc_sh.at[dst_v.at[ks]],
                                          ssem.at[bs]).wait()

                @pl.when(k + GD < CPW)
                def _():
                    bg = (b + GD) % NB
                    pltpu.async_copy(y_sh.at[src_v.at[k + GD]], rbufs[bg],
                                     gsem.at[bg])

                pltpu.make_async_copy(y_sh.at[src_v.at[k]], rbufs[b],
                                      gsem.at[b]).wait()
                pltpu.async_copy(rbufs[b], acc_sh.at[dst_v.at[k]], ssem.at[b],
                                 add=True)

        for j in range(CPW - (NB - GD), CPW):
            b = j % NB
            pltpu.make_async_copy(rbufs[b], acc_sh.at[dst_v.at[j]],
                                  ssem.at[b]).wait()

        plsc.subcore_barrier()
        pltpu.sync_copy(acc_sh.at[pl.ds(row0, rpt)],
                        out_hbm.at[cid, pl.ds(row0, rpt)])

    return hop_kernel


def _tc_a(x, W, degt, NPAD, BR):
    N, DIN = x.shape
    DOUT = W.shape[1]

    def body(x_ref, w_ref, d_ref, z_ref, n_ref):
        deg = jnp.sum(d_ref[...], axis=0, keepdims=True)
        deg = jnp.maximum(deg, 1.0)
        nrm = jnp.transpose(lax.rsqrt(deg))
        y = jnp.dot(x_ref[...], w_ref[...], preferred_element_type=jnp.float32)
        npad = n_ref.shape[0]
        ypad = jnp.pad(y, ((0, npad - y.shape[0]), (0, 0)))
        z_ref[...] = ypad * nrm
        n_ref[...] = nrm

    return pl.pallas_call(
        body,
        out_shape=[
            jax.ShapeDtypeStruct((NPAD, DOUT), jnp.float32),
            jax.ShapeDtypeStruct((NPAD, 1), jnp.float32),
        ],
    )(x, W, degt)


def _tc_combine(p, nrm, BR, power, relu, out_rows):
    _, NPAD, D = p.shape

    def body(p_ref, n_ref, o_ref):
        s = p_ref[0] + p_ref[1]
        n = n_ref[...]
        scale = n * n if power == 2 else n
        z = s * scale
        if relu:
            z = jnp.maximum(z, 0.0)
        o_ref[...] = z

    def body2(p_ref, n_ref, o_ref):
        s = p_ref[0, :out_rows] + p_ref[1, :out_rows]
        n = n_ref[:out_rows]
        scale = n * n if power == 2 else n
        z = s * scale
        if relu:
            z = jnp.maximum(z, 0.0)
        o_ref[...] = z

    return pl.pallas_call(
        body2,
        out_shape=jax.ShapeDtypeStruct((out_rows, D), jnp.float32),
    )(p, nrm)


def kernel(x, edge_index, W):
    N, DIN = x.shape
    DOUT = W.shape[1]
    E = edge_index.shape[1]
    NPAD = -(-(N + 1) // (8 * NS)) * (8 * NS)

    epw0 = -(-E // NW)
    CPW = -(-epw0 // CH)
    CPW = -(-CPW // NB) * NB
    EPW = CPW * CH
    EPAD = NW * EPW

    src = edge_index[0]
    dst = edge_index[1]
    pad = EPAD - E
    srcp = jnp.concatenate([src, jnp.zeros((pad,), jnp.int32)]).reshape(NW, CPW, CH)
    dstp = jnp.concatenate([dst, jnp.full((pad,), N, jnp.int32)]).reshape(NW, CPW, CH)
    dst_flat = dstp.reshape(NW, EPW)
    zeros = jnp.zeros((NPAD, DOUT), jnp.float32)

    degp = _make_deg_kernel(N, NPAD, EPW)(dst_flat)

    BR = NPAD // NS
    z0, nrm = _tc_a(x, W, degp, NPAD, BR)

    hop = _make_hop_kernel(N, NPAD, DOUT, CPW)
    p1 = hop(z0, srcp, dstp, zeros)
    z1 = _tc_combine(p1, nrm, BR, power=2, relu=False, out_rows=NPAD)
    p2 = hop(z1, srcp, dstp, zeros)
    return _tc_combine(p2, nrm, BR, power=1, relu=True, out_rows=N)
